# baseline (speedup 1.0000x reference)
import functools

import jax
import jax.numpy as jnp
from jax import lax
from jax.experimental import pallas as pl
from jax.experimental.pallas import tpu as pltpu
from jax.experimental.pallas import tpu_sc as plsc

NUM_WORKERS = 32
CHUNK = 32


def _make_emb_kernel(n_total, n_chunks, hidden):
    n_per_w = n_chunks * CHUNK
    mesh = plsc.VectorSubcoreMesh(core_axis_name="c", subcore_axis_name="s")

    @functools.partial(
        pl.kernel,
        mesh=mesh,
        out_type=jax.ShapeDtypeStruct((n_total, hidden), jnp.float32),
        scratch_types=[
            pltpu.VMEM((n_chunks, CHUNK), jnp.int32),
            pltpu.VMEM((3, CHUNK, hidden), jnp.float32),
            pltpu.SemaphoreType.DMA,
            pltpu.SemaphoreType.DMA,
            pltpu.SemaphoreType.DMA,
            pltpu.SemaphoreType.DMA,
            pltpu.SemaphoreType.DMA,
            pltpu.SemaphoreType.DMA,
        ],
    )
    def emb(idx_hbm, table_hbm, out_hbm, idx_v, rows_v, gsem0, gsem1, gsem2,
            osem0, osem1, osem2):
        gsems = (gsem0, gsem1, gsem2)
        osems = (osem0, osem1, osem2)
        wid = lax.axis_index("s") * 2 + lax.axis_index("c")
        base = wid * n_per_w

        def gather_start(c, slot):
            pltpu.async_copy(table_hbm.at[idx_v.at[c]], rows_v.at[slot],
                             gsems[slot])

        def gather_wait(c, slot):
            pltpu.make_async_copy(table_hbm.at[idx_v.at[c]], rows_v.at[slot],
                                  gsems[slot]).wait()

        def out_start(c, slot):
            pltpu.async_copy(rows_v.at[slot],
                             out_hbm.at[pl.ds(base + c * CHUNK, CHUNK)],
                             osems[slot])

        def out_wait(c, slot):
            pltpu.make_async_copy(rows_v.at[slot],
                                  out_hbm.at[pl.ds(base + c * CHUNK, CHUNK)],
                                  osems[slot]).wait()

        pltpu.sync_copy(idx_hbm.at[wid], idx_v)

        gather_start(0, 0)
        gather_start(1, 1)
        gather_start(2, 2)
        gather_wait(0, 0)
        out_start(0, 0)
        out_wait(0, 0)
        gather_start(3, 0)
        gather_wait(1, 1)
        out_start(1, 1)
        out_wait(1, 1)
        gather_start(4, 1)
        gather_wait(2, 2)
        out_start(2, 2)

        def step(i, _):
            for k in (0, 1, 2):
                c = 3 * i + 3 + k
                slot = k
                nslot = (k + 2) % 3
                out_wait(c - 1, nslot)
                gather_start(c + 2, nslot)
                gather_wait(c, slot)
                out_start(c, slot)
            return _

        lax.fori_loop(0, (n_chunks - 5) // 3, step, None)

        for c in (n_chunks - 2, n_chunks - 1):
            gather_wait(c, c % 3)
            out_start(c, c % 3)
        for c in (n_chunks - 3, n_chunks - 2, n_chunks - 1):
            out_wait(c, c % 3)

    return emb


def kernel(input_ids, word_embeddings):
    batch, seq = input_ids.shape
    vocab, hidden = word_embeddings.shape
    n_total = batch * seq
    n_per_w = n_total // NUM_WORKERS
    n_chunks = n_per_w // CHUNK

    idx = input_ids.reshape(NUM_WORKERS, n_chunks, CHUNK).astype(jnp.int32)
    out = _make_emb_kernel(n_total, n_chunks, hidden)(idx, word_embeddings)
    return out.reshape(batch, seq, hidden)
